# baseline (speedup 1.0000x reference)
import functools

import jax
import jax.numpy as jnp
from jax import lax
from jax.experimental import pallas as pl
from jax.experimental.pallas import tpu as pltpu
from jax.experimental.pallas import tpu_sc as plsc

_NC, _NS = 2, 16
_NW = _NC * _NS
_CH = 128


def _sc_gather(nodes, idxs):
    dw = nodes.shape[1]
    nseg = len(idxs)
    pers = [ix.shape[0] // _NW for ix in idxs]
    K = 2
    mesh = plsc.VectorSubcoreMesh(core_axis_name="c", subcore_axis_name="s")

    @functools.partial(
        pl.kernel,
        out_type=[jax.ShapeDtypeStruct((ix.shape[0], dw), jnp.float32)
                  for ix in idxs],
        mesh=mesh,
        scratch_types=[pltpu.VMEM((2, K, _CH), jnp.int32),
                       pltpu.VMEM((2, K * _CH, dw), jnp.float32),
                       pltpu.SemaphoreType.DMA,
                       pltpu.SemaphoreType.DMA],
    )
    def k(*refs):
        nodes_h = refs[0]
        idx_hs = refs[1:1 + nseg]
        out_hs = refs[1 + nseg:1 + 2 * nseg]
        idx_v, rows_v, sem0, sem1 = refs[1 + 2 * nseg:]
        sems = (sem0, sem1)
        wid = lax.axis_index("s") * _NC + lax.axis_index("c")

        def seg(idx_h, out_h, per):
            base = wid * per
            kk = min(K, per // _CH)
            ng = per // _CH // kk

            def fire(g, p):
                for j in range(kk):
                    off = base + (g * kk + j) * _CH
                    pltpu.sync_copy(idx_h.at[pl.ds(off, _CH)], idx_v.at[p, j])
                    pltpu.async_copy(nodes_h.at[idx_v.at[p, j]],
                                     rows_v.at[p, pl.ds(j * _CH, _CH)],
                                     sems[p])

            def drain(g, p):
                for j in range(kk):
                    pltpu.make_async_copy(nodes_h.at[idx_v.at[p, j]],
                                          rows_v.at[p, pl.ds(j * _CH, _CH)],
                                          sems[p]).wait()
                pltpu.sync_copy(rows_v.at[p, pl.ds(0, kk * _CH)],
                                out_h.at[pl.ds(base + g * kk * _CH, kk * _CH)])

            fire(0, 0)
            for g in range(1, ng):
                fire(g, g % 2)
                drain(g - 1, (g - 1) % 2)
            drain(ng - 1, (ng - 1) % 2)

        for idx_h, out_h, per in zip(idx_hs, out_hs, pers):
            seg(idx_h, out_h, per)

    return k(nodes, *idxs)


def _sc_scatter(msgs, dsts, B, S, d):
    nseg = len(msgs)
    halfs = [dx.shape[0] // _NC for dx in dsts]
    pers = [h // _NS for h in halfs]
    rows_sp = (B // _NC) * S
    per_out = rows_sp // _NS
    mesh = plsc.VectorSubcoreMesh(core_axis_name="c", subcore_axis_name="s")

    K = 2

    @functools.partial(
        pl.kernel,
        out_type=jax.ShapeDtypeStruct((B * S, d), jnp.float32),
        mesh=mesh,
        scratch_types=[pltpu.VMEM((2, K, _CH), jnp.int32),
                       pltpu.VMEM((2, K * _CH, d), jnp.float32),
                       pltpu.VMEM_SHARED((rows_sp, d), jnp.float32),
                       pltpu.SemaphoreType.DMA,
                       pltpu.SemaphoreType.DMA],
    )
    def k(*refs):
        m_hs = refs[:nseg]
        d_hs = refs[nseg:2 * nseg]
        agg_h = refs[2 * nseg]
        idx_v, mbuf, agg_sp, sem0, sem1 = refs[2 * nseg + 1:]
        sems = (sem0, sem1)
        cid = lax.axis_index("c")
        sid = lax.axis_index("s")

        zeros16 = jnp.zeros((16,), jnp.float32)

        @pl.loop(0, _CH)
        def _(r):
            for cc in range(d // 16):
                mbuf[0, r, pl.ds(cc * 16, 16)] = zeros16

        for kk in range(per_out // _CH):
            pltpu.sync_copy(mbuf.at[0, pl.ds(0, _CH)],
                            agg_sp.at[pl.ds(sid * per_out + kk * _CH, _CH)])
        plsc.subcore_barrier()

        def seg(m_h, idx_h, base, nch):
            kk = min(K, nch)
            ng = nch // kk

            def fire(g, p):
                for j in range(kk):
                    off = base + (g * kk + j) * _CH
                    pltpu.sync_copy(idx_h.at[pl.ds(off, _CH)], idx_v.at[p, j])
                    pltpu.async_copy(m_h.at[pl.ds(off, _CH)],
                                     mbuf.at[p, pl.ds(j * _CH, _CH)], sems[p])

            def drain(g, p):
                for j in range(kk):
                    off = base + (g * kk + j) * _CH
                    pltpu.make_async_copy(m_h.at[pl.ds(off, _CH)],
                                          mbuf.at[p, pl.ds(j * _CH, _CH)],
                                          sems[p]).wait()
                    pltpu.sync_copy(mbuf.at[p, pl.ds(j * _CH, _CH)],
                                    agg_sp.at[idx_v.at[p, j]], add=True)

            fire(0, 0)
            for g in range(1, ng):
                fire(g, g % 2)
                drain(g - 1, (g - 1) % 2)
            drain(ng - 1, (ng - 1) % 2)

        for m_h, d_h, half, per in zip(m_hs, d_hs, halfs, pers):
            seg(m_h, d_h, cid * half + sid * per, per // _CH)
        plsc.subcore_barrier()

        pltpu.sync_copy(agg_sp.at[pl.ds(sid * per_out, per_out)],
                        agg_h.at[pl.ds(cid * rows_sp + sid * per_out, per_out)])

    return k(*msgs, *dsts)


def _tc_tables(pred_emb, role_emb, W1, b1r):
    d = pred_emb.shape[1]
    P = pred_emb.shape[0]
    NRE = role_emb.shape[0]
    h2 = W1.shape[1]

    def body(pe_ref, re_ref, w1_ref, b1_ref, pb_ref, rec_ref):
        pb_ref[...] = jnp.dot(pe_ref[...], w1_ref[d:2 * d, :],
                              preferred_element_type=jnp.float32) + b1_ref[...]
        rec_ref[...] = jnp.dot(re_ref[...], w1_ref[2 * d:, :],
                               preferred_element_type=jnp.float32)

    return pl.pallas_call(
        body,
        out_shape=[jax.ShapeDtypeStruct((P, h2), jnp.float32),
                   jax.ShapeDtypeStruct((NRE, h2), jnp.float32)],
    )(pred_emb, role_emb, W1, b1r)


def _tc_msgs(lp, G0, G1, pred3, PB, W1, W2, b2r, BE):
    NE, d = G0.shape
    NB = NE // BE
    P, h2 = PB.shape

    def body(lp_ref, g0_ref, g1_ref, pr_ref, pb_ref, w1_ref, w2_ref,
             b2_ref, m01_ref, m10_ref):
        g0 = g0_ref[...]
        g1 = g1_ref[...]
        wa = w1_ref[:d, :]
        wc = w1_ref[2 * d:, :]
        x0a = jnp.dot(g0, wa, preferred_element_type=jnp.float32)
        x0c = jnp.dot(g0, wc, preferred_element_type=jnp.float32)
        x1a = jnp.dot(g1, wa, preferred_element_type=jnp.float32)
        x1c = jnp.dot(g1, wc, preferred_element_type=jnp.float32)
        pr = pr_ref[0, 0, :]
        iota = lax.broadcasted_iota(jnp.int32, (BE, P), 1)
        oh = (pr[:, None] == iota).astype(jnp.float32)
        peb = jnp.dot(oh, pb_ref[...], preferred_element_type=jnp.float32)
        w2 = w2_ref[...]
        b2v = b2_ref[...]
        h01 = x0a + peb + x1c
        h10 = x1a + peb + x0c
        m01_ref[...] = jnp.dot(jax.nn.gelu(h01), w2,
                               preferred_element_type=jnp.float32) + b2v
        mask = (pr != lp_ref[0]).astype(jnp.float32)[:, None]
        m10_ref[...] = (jnp.dot(jax.nn.gelu(h10), w2,
                                preferred_element_type=jnp.float32) + b2v) * mask

    return pl.pallas_call(
        body,
        grid=(NB,),
        in_specs=[
            pl.BlockSpec(memory_space=pltpu.SMEM),
            pl.BlockSpec((BE, d), lambda i: (i, 0)),
            pl.BlockSpec((BE, d), lambda i: (i, 0)),
            pl.BlockSpec((1, 1, BE), lambda i: (i, 0, 0)),
            pl.BlockSpec((P, h2), lambda i: (0, 0)),
            pl.BlockSpec((3 * d, h2), lambda i: (0, 0)),
            pl.BlockSpec((h2, d), lambda i: (0, 0)),
            pl.BlockSpec((1, d), lambda i: (0, 0)),
        ],
        out_specs=[pl.BlockSpec((BE, d), lambda i: (i, 0)),
                   pl.BlockSpec((BE, d), lambda i: (i, 0))],
        out_shape=[jax.ShapeDtypeStruct((NE, d), jnp.float32),
                   jax.ShapeDtypeStruct((NE, d), jnp.float32)],
    )(lp, G0, G1, pred3, PB, W1, W2, b2r)


def _tc_rmsgs(RN, ridx3, REc2, W1, W2, b2r, BR):
    NR, d = RN.shape
    NB = NR // BR
    NRE, h2 = REc2.shape

    def body(rn_ref, ri_ref, rec_ref, w1_ref, w2_ref, b2_ref, out_ref):
        rn = rn_ref[...]
        wa = w1_ref[:d, :]
        xra = jnp.dot(rn, wa, preferred_element_type=jnp.float32)
        ri = ri_ref[0, 0, :] + 1
        iota = lax.broadcasted_iota(jnp.int32, (BR, NRE), 1)
        oh = (ri[:, None] == iota).astype(jnp.float32)
        xrc = jnp.dot(oh, rec_ref[...], preferred_element_type=jnp.float32)
        h = xra + xrc
        out_ref[...] = jnp.dot(jax.nn.gelu(h), w2_ref[...],
                               preferred_element_type=jnp.float32) + b2_ref[...]

    return pl.pallas_call(
        body,
        grid=(NB,),
        in_specs=[
            pl.BlockSpec((BR, d), lambda i: (i, 0)),
            pl.BlockSpec((1, 1, BR), lambda i: (i, 0, 0)),
            pl.BlockSpec((NRE, h2), lambda i: (0, 0)),
            pl.BlockSpec((3 * d, h2), lambda i: (0, 0)),
            pl.BlockSpec((h2, d), lambda i: (0, 0)),
            pl.BlockSpec((1, d), lambda i: (0, 0)),
        ],
        out_specs=pl.BlockSpec((BR, d), lambda i: (i, 0)),
        out_shape=jax.ShapeDtypeStruct((NR, d), jnp.float32),
    )(RN, ridx3, REc2, W1, W2, b2r)


def _tc_final(nodes, AGG1, AGG2, ln_g, ln_b, Wl1, bl1, Wl2, bl2, B):
    S, d = nodes.shape
    dl = Wl1.shape[1]

    def body(n_ref, a_ref, a2_ref, g_ref, b_ref, wl1_ref, bl1_ref, wl2_ref,
             bl2_ref, out_ref):
        x = n_ref[...] + a_ref[...] + a2_ref[...]
        mu = jnp.mean(x, axis=1, keepdims=True)
        var = jnp.mean((x - mu) ** 2, axis=1, keepdims=True)
        h = (x - mu) / jnp.sqrt(var + 1e-5) * g_ref[...] + b_ref[...]
        pooled = jnp.mean(h, axis=0, keepdims=True)
        z = jnp.dot(pooled, wl1_ref[...],
                    preferred_element_type=jnp.float32) + bl1_ref[...]
        lat = jnp.dot(jax.nn.gelu(z), wl2_ref[...],
                      preferred_element_type=jnp.float32) + bl2_ref[...]
        out_ref[pl.ds(pl.program_id(0), 1), :] = lat

    return pl.pallas_call(
        body,
        grid=(B,),
        in_specs=[
            pl.BlockSpec((S, d), lambda i: (0, 0)),
            pl.BlockSpec((S, d), lambda i: (i, 0)),
            pl.BlockSpec((S, d), lambda i: (i, 0)),
            pl.BlockSpec((1, d), lambda i: (0, 0)),
            pl.BlockSpec((1, d), lambda i: (0, 0)),
            pl.BlockSpec((d, dl), lambda i: (0, 0)),
            pl.BlockSpec((1, dl), lambda i: (0, 0)),
            pl.BlockSpec((dl, dl), lambda i: (0, 0)),
            pl.BlockSpec((1, dl), lambda i: (0, 0)),
        ],
        out_specs=pl.BlockSpec((B, dl), lambda i: (0, 0)),
        out_shape=jax.ShapeDtypeStruct((B, dl), jnp.float32),
    )(nodes, AGG1, AGG2, ln_g, ln_b, Wl1, bl1, Wl2, bl2)


def kernel(pos_emb, pred_emb, role_emb, W1, b1, W2, b2, ln_g, ln_b,
           Wl1, bl1, Wl2, bl2,
           edge_pred, edge_a0, edge_a1, role_tok, role_idx,
           seq_len, link_pred, role_pred_local):
    S, d = pos_emb.shape
    B, E = edge_pred.shape
    R = role_tok.shape[1]
    h2 = W1.shape[1]

    nodes = lax.dynamic_slice_in_dim(pos_emb, seq_len - S, S)

    E1 = E // 2
    lp = jnp.asarray(link_pred, jnp.int32).reshape(1)
    b1r = b1.reshape(1, -1)
    b2r = b2.reshape(1, -1)
    offs = ((jnp.arange(B, dtype=jnp.int32) % (B // _NC)) * S)[:, None]
    BE = 1024

    a0h = [edge_a0[:, h * E1:(h + 1) * E1] for h in range(2)]
    a1h = [edge_a1[:, h * E1:(h + 1) * E1] for h in range(2)]
    prh = [edge_pred[:, h * E1:(h + 1) * E1] for h in range(2)]

    G0a, G1a, RN = _sc_gather(nodes, [a0h[0].reshape(-1), a1h[0].reshape(-1),
                                      role_tok.reshape(-1)])
    G0b, G1b = _sc_gather(nodes, [a0h[1].reshape(-1), a1h[1].reshape(-1)])

    PB, REc = _tc_tables(pred_emb, role_emb, W1, b1r)
    rpeb = lax.dynamic_slice(PB, (role_pred_local, 0), (1, h2))
    REc2 = REc + rpeb

    pred3a = prh[0].reshape(B * E1 // BE, 1, BE)
    pred3b = prh[1].reshape(B * E1 // BE, 1, BE)
    M01a, M10a = _tc_msgs(lp, G0a, G1a, pred3a, PB, W1, W2, b2r, BE)

    BR = R
    ridx3 = role_idx.reshape(B * R // BR, 1, BR)
    RMSG = _tc_rmsgs(RN, ridx3, REc2, W1, W2, b2r, BR)

    M01b, M10b = _tc_msgs(lp, G0b, G1b, pred3b, PB, W1, W2, b2r, BE)

    d1h = [(a1h[h] + offs).reshape(-1) for h in range(2)]
    d0h = [(a0h[h] + offs).reshape(-1) for h in range(2)]
    dr = (role_tok + offs).reshape(-1)
    AGG1 = _sc_scatter([M01a, M10a], [d1h[0], d0h[0]], B, S, d)
    AGG2 = _sc_scatter([M01b, M10b, RMSG], [d1h[1], d0h[1], dr], B, S, d)

    return _tc_final(nodes, AGG1, AGG2, ln_g.reshape(1, -1),
                     ln_b.reshape(1, -1), Wl1, bl1.reshape(1, -1), Wl2,
                     bl2.reshape(1, -1), B)


# speedup vs baseline: 1.0259x; 1.0259x over previous
import functools

import jax
import jax.numpy as jnp
from jax import lax
from jax.experimental import pallas as pl
from jax.experimental.pallas import tpu as pltpu
from jax.experimental.pallas import tpu_sc as plsc

_NC, _NS = 2, 16
_NW = _NC * _NS
_CH = 128


def _sc_gather(nodes, idxs):
    dw = nodes.shape[1]
    nseg = len(idxs)
    pers = [ix.shape[0] // _NW for ix in idxs]
    K = 2
    mesh = plsc.VectorSubcoreMesh(core_axis_name="c", subcore_axis_name="s")

    @functools.partial(
        pl.kernel,
        out_type=[jax.ShapeDtypeStruct((ix.shape[0], dw), jnp.float32)
                  for ix in idxs],
        mesh=mesh,
        scratch_types=[pltpu.VMEM((2, K, _CH), jnp.int32),
                       pltpu.VMEM((2, K * _CH, dw), jnp.float32),
                       pltpu.SemaphoreType.DMA,
                       pltpu.SemaphoreType.DMA],
    )
    def k(*refs):
        nodes_h = refs[0]
        idx_hs = refs[1:1 + nseg]
        out_hs = refs[1 + nseg:1 + 2 * nseg]
        idx_v, rows_v, sem0, sem1 = refs[1 + 2 * nseg:]
        sems = (sem0, sem1)
        wid = lax.axis_index("s") * _NC + lax.axis_index("c")

        def seg(idx_h, out_h, per):
            base = wid * per
            kk = min(K, per // _CH)
            ng = per // _CH // kk

            def fire(g, p):
                for j in range(kk):
                    off = base + (g * kk + j) * _CH
                    pltpu.sync_copy(idx_h.at[pl.ds(off, _CH)], idx_v.at[p, j])
                    pltpu.async_copy(nodes_h.at[idx_v.at[p, j]],
                                     rows_v.at[p, pl.ds(j * _CH, _CH)],
                                     sems[p])

            def drain(g, p):
                for j in range(kk):
                    pltpu.make_async_copy(nodes_h.at[idx_v.at[p, j]],
                                          rows_v.at[p, pl.ds(j * _CH, _CH)],
                                          sems[p]).wait()
                pltpu.sync_copy(rows_v.at[p, pl.ds(0, kk * _CH)],
                                out_h.at[pl.ds(base + g * kk * _CH, kk * _CH)])

            fire(0, 0)
            for g in range(1, ng):
                fire(g, g % 2)
                drain(g - 1, (g - 1) % 2)
            drain(ng - 1, (ng - 1) % 2)

        for idx_h, out_h, per in zip(idx_hs, out_hs, pers):
            seg(idx_h, out_h, per)

    return k(nodes, *idxs)


def _sc_scatter(msgs, dsts, B, S, d):
    nseg = len(msgs)
    halfs = [dx.shape[0] // _NC for dx in dsts]
    pers = [h // _NS for h in halfs]
    rows_sp = (B // _NC) * S
    per_out = rows_sp // _NS
    mesh = plsc.VectorSubcoreMesh(core_axis_name="c", subcore_axis_name="s")

    K = 2

    @functools.partial(
        pl.kernel,
        out_type=jax.ShapeDtypeStruct((B * S, d), jnp.float32),
        mesh=mesh,
        scratch_types=[pltpu.VMEM((2, K, _CH), jnp.int32),
                       pltpu.VMEM((2, K * _CH, d), jnp.float32),
                       pltpu.VMEM_SHARED((rows_sp, d), jnp.float32),
                       pltpu.SemaphoreType.DMA,
                       pltpu.SemaphoreType.DMA],
    )
    def k(*refs):
        m_hs = refs[:nseg]
        d_hs = refs[nseg:2 * nseg]
        agg_h = refs[2 * nseg]
        idx_v, mbuf, agg_sp, sem0, sem1 = refs[2 * nseg + 1:]
        sems = (sem0, sem1)
        cid = lax.axis_index("c")
        sid = lax.axis_index("s")

        zeros16 = jnp.zeros((16,), jnp.float32)

        @pl.loop(0, _CH)
        def _(r):
            for cc in range(d // 16):
                mbuf[0, r, pl.ds(cc * 16, 16)] = zeros16

        for kk in range(per_out // _CH):
            pltpu.sync_copy(mbuf.at[0, pl.ds(0, _CH)],
                            agg_sp.at[pl.ds(sid * per_out + kk * _CH, _CH)])
        plsc.subcore_barrier()

        def seg(m_h, idx_h, base, nch):
            kk = min(K, nch)
            ng = nch // kk

            def fire(g, p):
                for j in range(kk):
                    off = base + (g * kk + j) * _CH
                    pltpu.sync_copy(idx_h.at[pl.ds(off, _CH)], idx_v.at[p, j])
                    pltpu.async_copy(m_h.at[pl.ds(off, _CH)],
                                     mbuf.at[p, pl.ds(j * _CH, _CH)], sems[p])

            def drain(g, p):
                for j in range(kk):
                    off = base + (g * kk + j) * _CH
                    pltpu.make_async_copy(m_h.at[pl.ds(off, _CH)],
                                          mbuf.at[p, pl.ds(j * _CH, _CH)],
                                          sems[p]).wait()
                    pltpu.sync_copy(mbuf.at[p, pl.ds(j * _CH, _CH)],
                                    agg_sp.at[idx_v.at[p, j]], add=True)

            fire(0, 0)
            for g in range(1, ng):
                fire(g, g % 2)
                drain(g - 1, (g - 1) % 2)
            drain(ng - 1, (ng - 1) % 2)

        for m_h, d_h, half, per in zip(m_hs, d_hs, halfs, pers):
            seg(m_h, d_h, cid * half + sid * per, per // _CH)
        plsc.subcore_barrier()

        pltpu.sync_copy(agg_sp.at[pl.ds(sid * per_out, per_out)],
                        agg_h.at[pl.ds(cid * rows_sp + sid * per_out, per_out)])

    return k(*msgs, *dsts)


def _tc_tables(pred_emb, role_emb, W1, b1r):
    d = pred_emb.shape[1]
    P = pred_emb.shape[0]
    NRE = role_emb.shape[0]
    h2 = W1.shape[1]

    def body(pe_ref, re_ref, w1_ref, b1_ref, pb_ref, rec_ref):
        pb_ref[...] = jnp.dot(pe_ref[...], w1_ref[d:2 * d, :],
                              preferred_element_type=jnp.float32) + b1_ref[...]
        rec_ref[...] = jnp.dot(re_ref[...], w1_ref[2 * d:, :],
                               preferred_element_type=jnp.float32)

    return pl.pallas_call(
        body,
        out_shape=[jax.ShapeDtypeStruct((P, h2), jnp.float32),
                   jax.ShapeDtypeStruct((NRE, h2), jnp.float32)],
    )(pred_emb, role_emb, W1, b1r)


def _tc_msgs(lp, G0, G1, pred3, PB, W1, W2, b2r, BE):
    NE, d = G0.shape
    NB = NE // BE
    P, h2 = PB.shape

    def body(lp_ref, g0_ref, g1_ref, pr_ref, pb_ref, w1_ref, w2_ref,
             b2_ref, m01_ref, m10_ref):
        g0 = g0_ref[...]
        g1 = g1_ref[...]
        wa = w1_ref[:d, :]
        wc = w1_ref[2 * d:, :]
        x0a = jnp.dot(g0, wa, preferred_element_type=jnp.float32)
        x0c = jnp.dot(g0, wc, preferred_element_type=jnp.float32)
        x1a = jnp.dot(g1, wa, preferred_element_type=jnp.float32)
        x1c = jnp.dot(g1, wc, preferred_element_type=jnp.float32)
        pr = pr_ref[0, 0, :]
        iota = lax.broadcasted_iota(jnp.int32, (BE, P), 1)
        oh = (pr[:, None] == iota).astype(jnp.float32)
        peb = jnp.dot(oh, pb_ref[...], preferred_element_type=jnp.float32)
        w2 = w2_ref[...]
        b2v = b2_ref[...]
        h01 = x0a + peb + x1c
        h10 = x1a + peb + x0c
        m01_ref[...] = jnp.dot(jax.nn.gelu(h01), w2,
                               preferred_element_type=jnp.float32) + b2v
        mask = (pr != lp_ref[0]).astype(jnp.float32)[:, None]
        m10_ref[...] = (jnp.dot(jax.nn.gelu(h10), w2,
                                preferred_element_type=jnp.float32) + b2v) * mask

    return pl.pallas_call(
        body,
        grid=(NB,),
        in_specs=[
            pl.BlockSpec(memory_space=pltpu.SMEM),
            pl.BlockSpec((BE, d), lambda i: (i, 0)),
            pl.BlockSpec((BE, d), lambda i: (i, 0)),
            pl.BlockSpec((1, 1, BE), lambda i: (i, 0, 0)),
            pl.BlockSpec((P, h2), lambda i: (0, 0)),
            pl.BlockSpec((3 * d, h2), lambda i: (0, 0)),
            pl.BlockSpec((h2, d), lambda i: (0, 0)),
            pl.BlockSpec((1, d), lambda i: (0, 0)),
        ],
        out_specs=[pl.BlockSpec((BE, d), lambda i: (i, 0)),
                   pl.BlockSpec((BE, d), lambda i: (i, 0))],
        out_shape=[jax.ShapeDtypeStruct((NE, d), jnp.float32),
                   jax.ShapeDtypeStruct((NE, d), jnp.float32)],
    )(lp, G0, G1, pred3, PB, W1, W2, b2r)


def _tc_rmsgs(RN, ridx3, REc2, W1, W2, b2r, BR):
    NR, d = RN.shape
    NB = NR // BR
    NRE, h2 = REc2.shape

    def body(rn_ref, ri_ref, rec_ref, w1_ref, w2_ref, b2_ref, out_ref):
        rn = rn_ref[...]
        wa = w1_ref[:d, :]
        xra = jnp.dot(rn, wa, preferred_element_type=jnp.float32)
        ri = ri_ref[0, 0, :] + 1
        iota = lax.broadcasted_iota(jnp.int32, (BR, NRE), 1)
        oh = (ri[:, None] == iota).astype(jnp.float32)
        xrc = jnp.dot(oh, rec_ref[...], preferred_element_type=jnp.float32)
        h = xra + xrc
        out_ref[...] = jnp.dot(jax.nn.gelu(h), w2_ref[...],
                               preferred_element_type=jnp.float32) + b2_ref[...]

    return pl.pallas_call(
        body,
        grid=(NB,),
        in_specs=[
            pl.BlockSpec((BR, d), lambda i: (i, 0)),
            pl.BlockSpec((1, 1, BR), lambda i: (i, 0, 0)),
            pl.BlockSpec((NRE, h2), lambda i: (0, 0)),
            pl.BlockSpec((3 * d, h2), lambda i: (0, 0)),
            pl.BlockSpec((h2, d), lambda i: (0, 0)),
            pl.BlockSpec((1, d), lambda i: (0, 0)),
        ],
        out_specs=pl.BlockSpec((BR, d), lambda i: (i, 0)),
        out_shape=jax.ShapeDtypeStruct((NR, d), jnp.float32),
    )(RN, ridx3, REc2, W1, W2, b2r)


def _tc_final(nodes, AGGs, ln_g, ln_b, Wl1, bl1, Wl2, bl2, B):
    S, d = nodes.shape
    dl = Wl1.shape[1]
    na = len(AGGs)

    def body(*refs):
        n_ref = refs[0]
        a_refs = refs[1:1 + na]
        g_ref, b_ref, wl1_ref, bl1_ref, wl2_ref, bl2_ref, out_ref = refs[1 + na:]
        x = n_ref[...]
        for a_ref in a_refs:
            x = x + a_ref[...]
        mu = jnp.mean(x, axis=1, keepdims=True)
        var = jnp.mean((x - mu) ** 2, axis=1, keepdims=True)
        h = (x - mu) / jnp.sqrt(var + 1e-5) * g_ref[...] + b_ref[...]
        pooled = jnp.mean(h, axis=0, keepdims=True)
        z = jnp.dot(pooled, wl1_ref[...],
                    preferred_element_type=jnp.float32) + bl1_ref[...]
        lat = jnp.dot(jax.nn.gelu(z), wl2_ref[...],
                      preferred_element_type=jnp.float32) + bl2_ref[...]
        out_ref[pl.ds(pl.program_id(0), 1), :] = lat

    return pl.pallas_call(
        body,
        grid=(B,),
        in_specs=[pl.BlockSpec((S, d), lambda i: (0, 0))]
        + [pl.BlockSpec((S, d), lambda i: (i, 0))] * na
        + [
            pl.BlockSpec((1, d), lambda i: (0, 0)),
            pl.BlockSpec((1, d), lambda i: (0, 0)),
            pl.BlockSpec((d, dl), lambda i: (0, 0)),
            pl.BlockSpec((1, dl), lambda i: (0, 0)),
            pl.BlockSpec((dl, dl), lambda i: (0, 0)),
            pl.BlockSpec((1, dl), lambda i: (0, 0)),
        ],
        out_specs=pl.BlockSpec((B, dl), lambda i: (0, 0)),
        out_shape=jax.ShapeDtypeStruct((B, dl), jnp.float32),
    )(nodes, *AGGs, ln_g, ln_b, Wl1, bl1, Wl2, bl2)


def kernel(pos_emb, pred_emb, role_emb, W1, b1, W2, b2, ln_g, ln_b,
           Wl1, bl1, Wl2, bl2,
           edge_pred, edge_a0, edge_a1, role_tok, role_idx,
           seq_len, link_pred, role_pred_local):
    S, d = pos_emb.shape
    B, E = edge_pred.shape
    R = role_tok.shape[1]
    h2 = W1.shape[1]

    nodes = lax.dynamic_slice_in_dim(pos_emb, seq_len - S, S)

    NSP = 4
    E1 = E // NSP
    lp = jnp.asarray(link_pred, jnp.int32).reshape(1)
    b1r = b1.reshape(1, -1)
    b2r = b2.reshape(1, -1)
    offs = ((jnp.arange(B, dtype=jnp.int32) % (B // _NC)) * S)[:, None]
    BE = 1024

    a0h = [edge_a0[:, h * E1:(h + 1) * E1] for h in range(NSP)]
    a1h = [edge_a1[:, h * E1:(h + 1) * E1] for h in range(NSP)]
    prh = [edge_pred[:, h * E1:(h + 1) * E1] for h in range(NSP)]

    G = []
    G0a, G1a, RN = _sc_gather(nodes, [a0h[0].reshape(-1), a1h[0].reshape(-1),
                                      role_tok.reshape(-1)])
    G.append((G0a, G1a))
    for h in range(1, NSP):
        G.append(_sc_gather(nodes, [a0h[h].reshape(-1), a1h[h].reshape(-1)]))

    PB, REc = _tc_tables(pred_emb, role_emb, W1, b1r)
    rpeb = lax.dynamic_slice(PB, (role_pred_local, 0), (1, h2))
    REc2 = REc + rpeb

    M = []
    for h in range(NSP):
        pred3 = prh[h].reshape(B * E1 // BE, 1, BE)
        M.append(_tc_msgs(lp, G[h][0], G[h][1], pred3, PB, W1, W2, b2r, BE))
        if h == 0:
            BR = R
            ridx3 = role_idx.reshape(B * R // BR, 1, BR)
            RMSG = _tc_rmsgs(RN, ridx3, REc2, W1, W2, b2r, BR)

    d1h = [(a1h[h] + offs).reshape(-1) for h in range(NSP)]
    d0h = [(a0h[h] + offs).reshape(-1) for h in range(NSP)]
    dr = (role_tok + offs).reshape(-1)
    AGGs = []
    for h in range(NSP):
        msgs = [M[h][0], M[h][1]]
        dsts = [d1h[h], d0h[h]]
        if h == NSP - 1:
            msgs.append(RMSG)
            dsts.append(dr)
        AGGs.append(_sc_scatter(msgs, dsts, B, S, d))

    return _tc_final(nodes, AGGs, ln_g.reshape(1, -1),
                     ln_b.reshape(1, -1), Wl1, bl1.reshape(1, -1), Wl2,
                     bl2.reshape(1, -1), B)
